# Initial kernel scaffold; baseline (speedup 1.0000x reference)
import functools

import jax
import jax.numpy as jnp
from jax import lax
from jax.experimental import pallas as pl
from jax.experimental.pallas import tpu as pltpu
from jax.experimental.pallas import tpu_sc as plsc

B = 16384
EM = 16
HIST = 200
NUM_SPARSE = 13
DENSE = 17
FEAT_SC = NUM_SPARSE * EM + EM

NW = 32
S_PER_W = B // NW
G = 16
IDX_PER_G = G * HIST
ROWS_PER_G = IDX_PER_G // 128
N_GROUPS = S_PER_W // G


def _sc_embed_fn(uch_hbm, sp_hbm, t0, t1, t2, t3, t4, t5, t6, t7, t8, t9,
                 t10, t11, t12, feat_hbm, hidx_v, rows_v, fh_v, sidx_v,
                 srows_v, sem):
    tables = (t0, t1, t2, t3, t4, t5, t6, t7, t8, t9, t10, t11, t12)
    wid = lax.axis_index("subcore") * 2 + lax.axis_index("core")
    base = wid * S_PER_W
    hist_row0 = wid * (S_PER_W * HIST // 128)

    zero = jnp.zeros((16,), jnp.float32)

    @pl.loop(0, N_GROUPS)
    def _hist_group(g):
        pltpu.sync_copy(uch_hbm.at[pl.ds(hist_row0 + g * ROWS_PER_G,
                                         ROWS_PER_G)], hidx_v)

        @pl.loop(0, ROWS_PER_G)
        def _inc_row(r):
            @pl.loop(0, 128, step=16)
            def _inc(c):
                hidx_v[r, pl.ds(c, 16)] = hidx_v[r, pl.ds(c, 16)] + 1

        cps = [
            pltpu.async_copy(t0.at[hidx_v.at[j]],
                             rows_v.at[pl.ds(j * 128, 128)], sem)
            for j in range(ROWS_PER_G)
        ]
        for cp in cps:
            cp.wait()

        @pl.loop(0, G)
        def _sample(s):
            def body(j, accs):
                a0, a1 = accs
                o = s * HIST + j * 8
                for t in range(4):
                    a0 = a0 + rows_v[o + 2 * t]
                    a1 = a1 + rows_v[o + 2 * t + 1]
                return (a0, a1)

            a0, a1 = lax.fori_loop(0, HIST // 8, body, (zero, zero))
            fh_v[g * G + s] = a0 + a1

    pltpu.sync_copy(fh_v, feat_hbm.at[pl.ds(base, S_PER_W),
                                      pl.ds(NUM_SPARSE * EM, EM)])

    for i in range(NUM_SPARSE):
        pltpu.sync_copy(sp_hbm.at[i, pl.ds(wid * (S_PER_W // 128),
                                           S_PER_W // 128)], sidx_v)

        @pl.loop(0, S_PER_W // 128)
        def _sinc_row(r):
            @pl.loop(0, 128, step=16)
            def _sinc(c):
                sidx_v[r, pl.ds(c, 16)] = sidx_v[r, pl.ds(c, 16)] + 1

        cps = [
            pltpu.async_copy(tables[i].at[sidx_v.at[r]],
                             srows_v.at[pl.ds(r * 128, 128)], sem)
            for r in range(S_PER_W // 128)
        ]
        for cp in cps:
            cp.wait()
        pltpu.sync_copy(srows_v, feat_hbm.at[pl.ds(base, S_PER_W),
                                             pl.ds(i * EM, EM)])


def _mlp_fn(fs_ref, d_ref, w1s_ref, w1d_ref, b1_ref, w2_ref, b2_ref, w3_ref,
            b3_ref, o_ref):
    x = fs_ref[...]
    h = jnp.dot(x, w1s_ref[...], preferred_element_type=jnp.float32)
    h = h + jnp.dot(d_ref[...], w1d_ref[...],
                    preferred_element_type=jnp.float32)
    h = jnp.maximum(h + b1_ref[...], 0.0)
    h2 = jnp.dot(h, w2_ref[...], preferred_element_type=jnp.float32)
    h2 = jnp.maximum(h2 + b2_ref[...], 0.0)
    o_ref[...] = (jnp.dot(h2, w3_ref[...], preferred_element_type=jnp.float32)
                  + b3_ref[...])


def kernel(sparse_features, dense_features, user_click_history, tables,
           fc1_w, fc1_b, fc2_w, fc2_b, fc3_w, fc3_b):
    uch2 = user_click_history.reshape(B * HIST // 128, 128)
    sp3 = sparse_features.T.reshape(NUM_SPARSE, B // 128, 128)

    mesh = plsc.VectorSubcoreMesh(core_axis_name="core",
                                  subcore_axis_name="subcore")
    sc_embed = pl.kernel(
        _sc_embed_fn,
        out_type=jax.ShapeDtypeStruct((B, FEAT_SC), jnp.float32),
        mesh=mesh,
        scratch_types=[
            pltpu.VMEM((ROWS_PER_G, 128), jnp.int32),
            pltpu.VMEM((IDX_PER_G, EM), jnp.float32),
            pltpu.VMEM((S_PER_W, EM), jnp.float32),
            pltpu.VMEM((S_PER_W // 128, 128), jnp.int32),
            pltpu.VMEM((S_PER_W, EM), jnp.float32),
            pltpu.SemaphoreType.DMA,
        ],
    )
    feat = sc_embed(uch2, sp3, *tables)

    w1s = fc1_w[:, :FEAT_SC].T
    w1d = fc1_w[:, FEAT_SC:].T
    w2t = fc2_w.T
    w3t = fc3_w.T
    b1r = fc1_b.reshape(1, -1)
    b2r = fc2_b.reshape(1, -1)
    b3r = fc3_b.reshape(1, -1)

    BLK = 2048
    out = pl.pallas_call(
        _mlp_fn,
        grid=(B // BLK,),
        in_specs=[
            pl.BlockSpec((BLK, FEAT_SC), lambda i: (i, 0)),
            pl.BlockSpec((BLK, DENSE), lambda i: (i, 0)),
            pl.BlockSpec(w1s.shape, lambda i: (0, 0)),
            pl.BlockSpec(w1d.shape, lambda i: (0, 0)),
            pl.BlockSpec(b1r.shape, lambda i: (0, 0)),
            pl.BlockSpec(w2t.shape, lambda i: (0, 0)),
            pl.BlockSpec(b2r.shape, lambda i: (0, 0)),
            pl.BlockSpec(w3t.shape, lambda i: (0, 0)),
            pl.BlockSpec(b3r.shape, lambda i: (0, 0)),
        ],
        out_specs=pl.BlockSpec((BLK, 2), lambda i: (i, 0)),
        out_shape=jax.ShapeDtypeStruct((B, 2), jnp.float32),
    )(feat, dense_features, w1s, w1d, b1r, w2t, b2r, w3t, b3r)
    return out


# speedup vs baseline: 5.6777x; 5.6777x over previous
import jax
import jax.numpy as jnp
from jax import lax
from jax.experimental import pallas as pl
from jax.experimental.pallas import tpu as pltpu
from jax.experimental.pallas import tpu_sc as plsc

B = 16384
EM = 16
HIST = 200
NUM_SPARSE = 13
DENSE = 17
NUM_BLOCKS = NUM_SPARSE + 1
FEAT_SC = NUM_BLOCKS * EM

NW = 32
S_PER_W = B // NW
G = 16
IDX_PER_G = G * HIST
N_GROUPS = S_PER_W // G
S_CHUNKS = S_PER_W // 128


def _sc_embed_fn(uch_hbm, sp_hbm, t0, t1, t2, t3, t4, t5, t6, t7, t8, t9,
                 t10, t11, t12, out_hbm, hidx_v, rows_v, fh_v, sidx_v,
                 srows_v, scat_v, sem):
    tables = (t0, t1, t2, t3, t4, t5, t6, t7, t8, t9, t10, t11, t12)
    wid = lax.axis_index("subcore") * 2 + lax.axis_index("core")
    base = wid * S_PER_W

    zero = jnp.zeros((16,), jnp.float32)
    lane = lax.iota(jnp.int32, 16)

    def fill_scatter_idx(block):
        @pl.loop(0, S_CHUNKS)
        def _fr(r):
            @pl.loop(0, 128, step=16)
            def _fc(c):
                k = base + r * 128 + c + lane
                scat_v[r, pl.ds(c, 16)] = k * NUM_BLOCKS + block

    @pl.loop(0, N_GROUPS)
    def _hist_group(g):
        off = wid * (S_PER_W * HIST) + g * IDX_PER_G
        pltpu.sync_copy(uch_hbm.at[pl.ds(off, IDX_PER_G)], hidx_v)

        @pl.loop(0, IDX_PER_G, step=16)
        def _inc(c):
            hidx_v[pl.ds(c, 16)] = hidx_v[pl.ds(c, 16)] + 1

        cps = [
            pltpu.async_copy(t0.at[hidx_v.at[pl.ds(j * 128, 128)]],
                             rows_v.at[pl.ds(j * 128, 128)], sem)
            for j in range(IDX_PER_G // 128)
        ]
        for cp in cps:
            cp.wait()

        @pl.loop(0, G)
        def _sample(s):
            def body(j, accs):
                a0, a1 = accs
                o = s * HIST + j * 8
                for t in range(4):
                    a0 = a0 + rows_v[o + 2 * t]
                    a1 = a1 + rows_v[o + 2 * t + 1]
                return (a0, a1)

            a0, a1 = lax.fori_loop(0, HIST // 8, body, (zero, zero))
            fh_v[g * G + s] = a0 + a1

    fill_scatter_idx(NUM_SPARSE)
    cps = [
        pltpu.async_copy(fh_v.at[pl.ds(r * 128, 128)],
                         out_hbm.at[scat_v.at[r]], sem)
        for r in range(S_CHUNKS)
    ]
    for cp in cps:
        cp.wait()

    for i in range(NUM_SPARSE):
        pltpu.sync_copy(sp_hbm.at[pl.ds(i * B + base, S_PER_W)], sidx_v)

        @pl.loop(0, S_PER_W, step=16)
        def _sinc(c):
            sidx_v[pl.ds(c, 16)] = sidx_v[pl.ds(c, 16)] + 1

        cps = [
            pltpu.async_copy(tables[i].at[sidx_v.at[pl.ds(r * 128, 128)]],
                             srows_v.at[pl.ds(r * 128, 128)], sem)
            for r in range(S_CHUNKS)
        ]
        for cp in cps:
            cp.wait()

        fill_scatter_idx(i)
        cps = [
            pltpu.async_copy(srows_v.at[pl.ds(r * 128, 128)],
                             out_hbm.at[scat_v.at[r]], sem)
            for r in range(S_CHUNKS)
        ]
        for cp in cps:
            cp.wait()


def _mlp_fn(fs_ref, d_ref, w1s_ref, w1d_ref, b1_ref, w2_ref, b2_ref, w3_ref,
            b3_ref, o_ref):
    x = fs_ref[...]
    h = jnp.dot(x, w1s_ref[...], preferred_element_type=jnp.float32)
    h = h + jnp.dot(d_ref[...], w1d_ref[...],
                    preferred_element_type=jnp.float32)
    h = jnp.maximum(h + b1_ref[...], 0.0)
    h2 = jnp.dot(h, w2_ref[...], preferred_element_type=jnp.float32)
    h2 = jnp.maximum(h2 + b2_ref[...], 0.0)
    o_ref[...] = (jnp.dot(h2, w3_ref[...], preferred_element_type=jnp.float32)
                  + b3_ref[...])


def kernel(sparse_features, dense_features, user_click_history, tables,
           fc1_w, fc1_b, fc2_w, fc2_b, fc3_w, fc3_b):
    uch1 = user_click_history.reshape(-1)
    sp1 = sparse_features.T.reshape(-1)

    mesh = plsc.VectorSubcoreMesh(core_axis_name="core",
                                  subcore_axis_name="subcore")
    sc_embed = pl.kernel(
        _sc_embed_fn,
        out_type=jax.ShapeDtypeStruct((NUM_BLOCKS * B, EM), jnp.float32),
        mesh=mesh,
        scratch_types=[
            pltpu.VMEM((IDX_PER_G,), jnp.int32),
            pltpu.VMEM((IDX_PER_G, EM), jnp.float32),
            pltpu.VMEM((S_PER_W, EM), jnp.float32),
            pltpu.VMEM((S_PER_W,), jnp.int32),
            pltpu.VMEM((S_PER_W, EM), jnp.float32),
            pltpu.VMEM((S_CHUNKS, 128), jnp.int32),
            pltpu.SemaphoreType.DMA,
        ],
        compiler_params=pltpu.CompilerParams(use_tc_tiling_on_sc=False),
    )
    feat = sc_embed(uch1, sp1, *tables).reshape(B, FEAT_SC)

    w1s = fc1_w[:, :FEAT_SC].T
    w1d = fc1_w[:, FEAT_SC:].T
    w2t = fc2_w.T
    w3t = fc3_w.T
    b1r = fc1_b.reshape(1, -1)
    b2r = fc2_b.reshape(1, -1)
    b3r = fc3_b.reshape(1, -1)

    BLK = 2048
    out = pl.pallas_call(
        _mlp_fn,
        grid=(B // BLK,),
        in_specs=[
            pl.BlockSpec((BLK, FEAT_SC), lambda i: (i, 0)),
            pl.BlockSpec((BLK, DENSE), lambda i: (i, 0)),
            pl.BlockSpec(w1s.shape, lambda i: (0, 0)),
            pl.BlockSpec(w1d.shape, lambda i: (0, 0)),
            pl.BlockSpec(b1r.shape, lambda i: (0, 0)),
            pl.BlockSpec(w2t.shape, lambda i: (0, 0)),
            pl.BlockSpec(b2r.shape, lambda i: (0, 0)),
            pl.BlockSpec(w3t.shape, lambda i: (0, 0)),
            pl.BlockSpec(b3r.shape, lambda i: (0, 0)),
        ],
        out_specs=pl.BlockSpec((BLK, 2), lambda i: (i, 0)),
        out_shape=jax.ShapeDtypeStruct((B, 2), jnp.float32),
    )(feat, dense_features, w1s, w1d, b1r, w2t, b2r, w3t, b3r)
    return out
